# baseline (speedup 1.0000x reference)
import functools

import numpy as np
import jax
import jax.numpy as jnp
from jax.experimental import pallas as pl

_B, _N, _D, _R = 1, 2048, 1024, 8
_CB, _CIC, _KV = 64, 16, 64
_OP = (_D * _R) // _CB
_G = 8
_T = 256

_PERM = np.arange(_CB).reshape(8, 8).T.reshape(-1)

_DNT = (((1,), (1,)), ((), ()))
_DN = (((1,), (0,)), ((), ()))


def _prep(W, keys, vals):
    Wt = W[_PERM].reshape(_CB * _CIC, _D)
    kp = keys[_PERM].reshape(_G, 8, _KV, _CIC)
    eye = jnp.eye(8, dtype=W.dtype)
    KT = jnp.einsum('qjkg,ji->qikjg', kp, eye).reshape(_G, 8 * _KV, 8 * _CIC)
    kn = (kp ** 2).sum(-1).reshape(_G, 8 * _KV, 1)
    V = vals[_PERM].reshape(_G, 8 * _KV, _OP)
    return Wt, KT, kn, V


def _kmin(sc, ko):
    m = jnp.min(sc, axis=1, keepdims=True)
    return jnp.min(jnp.where(sc == m, ko, _KV), axis=1, keepdims=True)


def _body(x_ref, wa_ref, ka_ref, kna_ref, va_ref,
          wb_ref, kb_ref, knb_ref, vb_ref, out_ref):
    x = x_ref[...]
    pA = jax.lax.dot_general(x, wa_ref[...], _DNT,
                             preferred_element_type=jnp.float32)
    pB = jax.lax.dot_general(x, wb_ref[...], _DNT,
                             preferred_element_type=jnp.float32)
    ko = jax.lax.broadcasted_iota(jnp.int32, (8, _KV, _T), 1)
    t = None
    kminB = [None] * _G
    for q in range(_G):
        xq = x[:, q * 128:(q + 1) * 128]
        crossA = jax.lax.dot_general(ka_ref[q], pA[:, q * 128:(q + 1) * 128],
                                     _DNT, preferred_element_type=jnp.float32)
        scA = (kna_ref[q] - 2.0 * crossA).reshape(8, _KV, _T)
        kA = _kmin(scA, ko)
        PT = jax.lax.dot_general(va_ref[q], xq, _DNT,
                                 preferred_element_type=jnp.float32
                                 ).reshape(8, _KV, _T)
        s = jnp.sum(jnp.where(ko == kA, PT, 0.0), axis=1, keepdims=True)
        t = s if t is None else t + s
        crossB = jax.lax.dot_general(kb_ref[q], pB[:, q * 128:(q + 1) * 128],
                                     _DNT, preferred_element_type=jnp.float32)
        scB = (knb_ref[q] - 2.0 * crossB).reshape(8, _KV, _T)
        kminB[q] = _kmin(scB, ko)
    tT = jnp.transpose(t.reshape(8, _T))
    kmT = jnp.transpose(
        jnp.concatenate([k.reshape(8, _T) for k in kminB], axis=0))
    io = jax.lax.broadcasted_iota(jnp.int32, (_T, _KV), 1)
    for q in range(_G):
        w = jnp.concatenate(
            [jnp.where(io == kmT[:, q * 8 + j:q * 8 + j + 1],
                       jnp.broadcast_to(tT[:, j:j + 1], (_T, _KV)), 0.0)
             for j in range(8)], axis=1)
        out_ref[:, q * 128:(q + 1) * 128] = jax.lax.dot_general(
            w, vb_ref[q], _DN, preferred_element_type=jnp.float32)


@functools.partial(jax.jit, static_argnames=("interpret",))
def _run(x, W_A, keys_A, vals_A, W_B, keys_B, vals_B, interpret=False):
    WAt, KAT, knA, VA = _prep(W_A, keys_A, vals_A)
    WBt, KBT, knB, VB = _prep(W_B, keys_B, vals_B)
    full = lambda *s: pl.BlockSpec(s, lambda i: (0,) * len(s))
    out = pl.pallas_call(
        _body,
        grid=(_N // _T,),
        in_specs=[
            pl.BlockSpec((_T, _D), lambda i: (i, 0)),
            full(_CB * _CIC, _D),
            full(_G, 8 * _KV, 8 * _CIC),
            full(_G, 8 * _KV, 1),
            full(_G, 8 * _KV, _OP),
            full(_CB * _CIC, _D),
            full(_G, 8 * _KV, 8 * _CIC),
            full(_G, 8 * _KV, 1),
            full(_G, 8 * _KV, _OP),
        ],
        out_specs=pl.BlockSpec((_T, _D), lambda i: (i, 0)),
        out_shape=jax.ShapeDtypeStruct((_N, _D), jnp.float32),
        interpret=interpret,
    )(x.reshape(_N, _D), WAt, KAT, knA, VA, WBt, KBT, knB, VB)
    return out.reshape(_B, _N, _D)


def kernel(x, W_A, keys_A, vals_A, W_B, keys_B, vals_B):
    return _run(x, W_A, keys_A, vals_A, W_B, keys_B, vals_B)


# speedup vs baseline: 1.2812x; 1.2812x over previous
import functools

import numpy as np
import jax
import jax.numpy as jnp
from jax.experimental import pallas as pl

_B, _N, _D, _R = 1, 2048, 1024, 8
_CB, _CIC, _KV = 64, 16, 64
_OP = (_D * _R) // _CB
_G = 8
_T = 256

_PERM = np.arange(_CB).reshape(8, 8).T.reshape(-1)

_DNT = (((1,), (1,)), ((), ()))
_DN = (((1,), (0,)), ((), ()))


def _prep(W, keys, vals):
    Wt = W[_PERM].reshape(_CB * _CIC, _D)
    kp = keys[_PERM].reshape(_G, 8, _KV, _CIC)
    eye = jnp.eye(8, dtype=W.dtype)
    KT = jnp.einsum('qjkg,ji->qikjg', kp, eye).reshape(_G, 8 * _KV, 8 * _CIC)
    kn = (kp ** 2).sum(-1).reshape(_G, 8 * _KV, 1)
    V = vals[_PERM].reshape(_G, 8 * _KV, _OP)
    return Wt, KT, kn, V


def _kmin(sc, ko):
    m = jnp.min(sc, axis=1, keepdims=True)
    return jnp.min(jnp.where(sc == m, ko, _KV), axis=1, keepdims=True)


def _body(x_ref, wa_ref, ka_ref, kna_ref, va_ref,
          wb_ref, kb_ref, knb_ref, vbt_ref, out_ref):
    xt = jnp.transpose(x_ref[...])
    pTA = jnp.dot(wa_ref[...], xt, preferred_element_type=jnp.float32)
    pTB = jnp.dot(wb_ref[...], xt, preferred_element_type=jnp.float32)
    ko = jax.lax.broadcasted_iota(jnp.int32, (8, _KV, _T), 1)
    t = None
    kminB = [None] * _G
    for q in range(_G):
        xq = xt[q * 128:(q + 1) * 128, :]
        crossA = jnp.dot(ka_ref[q], pTA[q * 128:(q + 1) * 128, :],
                         preferred_element_type=jnp.float32)
        scA = (kna_ref[q] - 2.0 * crossA).reshape(8, _KV, _T)
        kA = _kmin(scA, ko)
        PT = jnp.dot(va_ref[q], xq,
                     preferred_element_type=jnp.float32).reshape(8, _KV, _T)
        s = jnp.sum(jnp.where(ko == kA, PT, 0.0), axis=1, keepdims=True)
        t = s if t is None else t + s
        crossB = jnp.dot(kb_ref[q], pTB[q * 128:(q + 1) * 128, :],
                         preferred_element_type=jnp.float32)
        scB = (knb_ref[q] - 2.0 * crossB).reshape(8, _KV, _T)
        kminB[q] = _kmin(scB, ko)
    for q in range(_G):
        w = jnp.where(ko == kminB[q], jnp.broadcast_to(t, ko.shape), 0.0)
        out_ref[:, q * 128:(q + 1) * 128] = jnp.transpose(jnp.dot(
            vbt_ref[q], w.reshape(8 * _KV, _T),
            preferred_element_type=jnp.float32))


@functools.partial(jax.jit, static_argnames=("interpret",))
def _run(x, W_A, keys_A, vals_A, W_B, keys_B, vals_B, interpret=False):
    WAt, KAT, knA, VA = _prep(W_A, keys_A, vals_A)
    WBt, KBT, knB, VB = _prep(W_B, keys_B, vals_B)
    VBT = VB.transpose(0, 2, 1)
    full = lambda *s: pl.BlockSpec(s, lambda i: (0,) * len(s))
    out = pl.pallas_call(
        _body,
        grid=(_N // _T,),
        in_specs=[
            pl.BlockSpec((_T, _D), lambda i: (i, 0)),
            full(_CB * _CIC, _D),
            full(_G, 8 * _KV, 8 * _CIC),
            full(_G, 8 * _KV, 1),
            full(_G, 8 * _KV, _OP),
            full(_CB * _CIC, _D),
            full(_G, 8 * _KV, 8 * _CIC),
            full(_G, 8 * _KV, 1),
            full(_G, _OP, 8 * _KV),
        ],
        out_specs=pl.BlockSpec((_T, _D), lambda i: (i, 0)),
        out_shape=jax.ShapeDtypeStruct((_N, _D), jnp.float32),
        interpret=interpret,
    )(x.reshape(_N, _D), WAt, KAT, knA, VA, WBt, KBT, knB, VBT)
    return out.reshape(_B, _N, _D)


def kernel(x, W_A, keys_A, vals_A, W_B, keys_B, vals_B):
    return _run(x, W_A, keys_A, vals_A, W_B, keys_B, vals_B)


# speedup vs baseline: 1.4954x; 1.1672x over previous
import functools

import numpy as np
import jax
import jax.numpy as jnp
from jax.experimental import pallas as pl

_B, _N, _D, _R = 1, 2048, 1024, 8
_CB, _CIC, _KV = 64, 16, 64
_OP = (_D * _R) // _CB
_G = 8
_T = 512

_PERM = np.arange(_CB).reshape(8, 8).T.reshape(-1)


def _prep(W, keys, vals):
    Wt = W.reshape(_CB * _CIC, _D)
    kp = keys.reshape(_G, 8, _KV, _CIC)
    eye = jnp.eye(8, dtype=W.dtype)
    KT = jnp.einsum('hjkg,ji->hjkig', kp, eye).reshape(_G, 8 * _KV, 8 * _CIC)
    kn = (kp ** 2).sum(-1).reshape(_G, 8 * _KV, 1)
    V = vals[_PERM].reshape(_G, 8 * _KV, _OP)
    return Wt, KT, kn, V


def _kmin(sc, ko):
    m = jnp.min(sc, axis=1, keepdims=True)
    return jnp.min(jnp.where(sc == m, ko, _KV), axis=1, keepdims=True)


def _body(x_ref, wa_ref, ka_ref, kna_ref, va_ref,
          wb_ref, kb_ref, knb_ref, vbt_ref, out_ref):
    xt = jnp.transpose(x_ref[...])
    pTA = jnp.dot(wa_ref[...], xt, preferred_element_type=jnp.float32)
    pTB = jnp.dot(wb_ref[...], xt, preferred_element_type=jnp.float32)
    ko = jax.lax.broadcasted_iota(jnp.int32, (8, _KV, _T), 1)
    kmA = [None] * _G
    kmB = [None] * _G
    for h in range(_G):
        crossA = jnp.dot(ka_ref[h], pTA[h * 128:(h + 1) * 128, :],
                         preferred_element_type=jnp.float32)
        kmA[h] = _kmin((kna_ref[h] - 2.0 * crossA).reshape(8, _KV, _T), ko)
        crossB = jnp.dot(kb_ref[h], pTB[h * 128:(h + 1) * 128, :],
                         preferred_element_type=jnp.float32)
        kmB[h] = _kmin((knb_ref[h] - 2.0 * crossB).reshape(8, _KV, _T), ko)
    t = None
    kBq = [None] * _G
    for q in range(_G):
        kAq = jnp.concatenate([kmA[r][q:q + 1] for r in range(8)], axis=0)
        kBq[q] = jnp.concatenate([kmB[r][q:q + 1] for r in range(8)], axis=0)
        PT = jnp.dot(va_ref[q], xt[q * 128:(q + 1) * 128, :],
                     preferred_element_type=jnp.float32).reshape(8, _KV, _T)
        s = jnp.sum(jnp.where(ko == kAq, PT, 0.0), axis=1, keepdims=True)
        t = s if t is None else t + s
    for q in range(_G):
        w = jnp.where(ko == kBq[q], jnp.broadcast_to(t, ko.shape), 0.0)
        out_ref[:, q * 128:(q + 1) * 128] = jnp.transpose(jnp.dot(
            vbt_ref[q], w.reshape(8 * _KV, _T),
            preferred_element_type=jnp.float32))


@functools.partial(jax.jit, static_argnames=("interpret",))
def _run(x, W_A, keys_A, vals_A, W_B, keys_B, vals_B, interpret=False):
    WAt, KAT, knA, VA = _prep(W_A, keys_A, vals_A)
    WBt, KBT, knB, VB = _prep(W_B, keys_B, vals_B)
    VBT = VB.transpose(0, 2, 1)
    full = lambda *s: pl.BlockSpec(s, lambda i: (0,) * len(s))
    out = pl.pallas_call(
        _body,
        grid=(_N // _T,),
        in_specs=[
            pl.BlockSpec((_T, _D), lambda i: (i, 0)),
            full(_CB * _CIC, _D),
            full(_G, 8 * _KV, 8 * _CIC),
            full(_G, 8 * _KV, 1),
            full(_G, 8 * _KV, _OP),
            full(_CB * _CIC, _D),
            full(_G, 8 * _KV, 8 * _CIC),
            full(_G, 8 * _KV, 1),
            full(_G, _OP, 8 * _KV),
        ],
        out_specs=pl.BlockSpec((_T, _D), lambda i: (i, 0)),
        out_shape=jax.ShapeDtypeStruct((_N, _D), jnp.float32),
        interpret=interpret,
    )(x.reshape(_N, _D), WAt, KAT, knA, VA, WBt, KBT, knB, VBT)
    return out.reshape(_B, _N, _D)


def kernel(x, W_A, keys_A, vals_A, W_B, keys_B, vals_B):
    return _run(x, W_A, keys_A, vals_A, W_B, keys_B, vals_B)
